# Initial kernel scaffold; baseline (speedup 1.0000x reference)
import jax
import jax.numpy as jnp
from jax.experimental import pallas as pl


def kernel(x, embedding):
    raise NotImplementedError("write your pallas kernel here")


# speedup vs baseline: 5.7615x; 5.7615x over previous
import functools

import jax
import jax.numpy as jnp
from jax import lax
from jax.experimental import pallas as pl
from jax.experimental.pallas import tpu as pltpu
from jax.experimental.pallas import tpu_sc as plsc

BATCH = 1024
SEQ = 200
NUM_TOKENS = 100000
EMBED_DIM = 128

N = BATCH * SEQ
NW = 32
PER_W = N // NW
CHUNK = 128
NCHUNK = PER_W // CHUNK

_mesh = plsc.VectorSubcoreMesh(core_axis_name="c", subcore_axis_name="s")


@functools.partial(
    pl.kernel,
    out_type=jax.ShapeDtypeStruct((N, EMBED_DIM), jnp.float32),
    mesh=_mesh,
    scratch_types=[
        pltpu.VMEM((PER_W,), jnp.int32),
        pltpu.VMEM((CHUNK, EMBED_DIM), jnp.float32),
        pltpu.SemaphoreType.DMA,
    ],
)
def _gather_kernel(table_hbm, idx_hbm, out_hbm, idx_v, rows_v, sem):
    wid = lax.axis_index("s") * 2 + lax.axis_index("c")
    base = wid * PER_W
    pltpu.sync_copy(idx_hbm.at[pl.ds(base, PER_W)], idx_v)

    def body(j, _):
        off = j * CHUNK
        idx_slice = idx_v.at[pl.ds(off, CHUNK)]
        pltpu.async_copy(table_hbm.at[idx_slice], rows_v, sem).wait()
        pltpu.sync_copy(rows_v, out_hbm.at[pl.ds(base + off, CHUNK)])
        return 0

    lax.fori_loop(0, NCHUNK, body, 0)


def kernel(x, embedding):
    x_flat = x.reshape(N).astype(jnp.int32)
    out = _gather_kernel(embedding, x_flat)
    return out.reshape(BATCH, SEQ, EMBED_DIM)


# speedup vs baseline: 7.2512x; 1.2586x over previous
import functools

import jax
import jax.numpy as jnp
from jax import lax
from jax.experimental import pallas as pl
from jax.experimental.pallas import tpu as pltpu
from jax.experimental.pallas import tpu_sc as plsc

BATCH = 1024
SEQ = 200
NUM_TOKENS = 100000
EMBED_DIM = 128

N = BATCH * SEQ
NW = 32
PER_W = N // NW
CHUNK = 128
NCHUNK = PER_W // CHUNK
NBUF = 2
NGRP = NCHUNK // NBUF

_mesh = plsc.VectorSubcoreMesh(core_axis_name="c", subcore_axis_name="s")


@functools.partial(
    pl.kernel,
    out_type=jax.ShapeDtypeStruct((N, EMBED_DIM), jnp.float32),
    mesh=_mesh,
    scratch_types=[
        pltpu.VMEM((PER_W,), jnp.int32),
        [pltpu.VMEM((CHUNK, EMBED_DIM), jnp.float32) for _ in range(NBUF)],
        [pltpu.SemaphoreType.DMA for _ in range(NBUF)],
        [pltpu.SemaphoreType.DMA for _ in range(NBUF)],
    ],
)
def _gather_kernel(table_hbm, idx_hbm, out_hbm, idx_v, bufs, gsems, ssems):
    wid = lax.axis_index("s") * 2 + lax.axis_index("c")
    base = wid * PER_W
    pltpu.sync_copy(idx_hbm.at[pl.ds(base, PER_W)], idx_v)

    def start_gather(j, b):
        idx_slice = idx_v.at[pl.ds(j * CHUNK, CHUNK)]
        pltpu.async_copy(table_hbm.at[idx_slice], bufs[b], gsems[b])

    def wait_gather(b):
        pltpu.make_async_copy(table_hbm.at[idx_v.at[pl.ds(0, CHUNK)]],
                              bufs[b], gsems[b]).wait()

    def start_store(j, b):
        pltpu.async_copy(bufs[b], out_hbm.at[pl.ds(base + j * CHUNK, CHUNK)],
                         ssems[b])

    def wait_store(b):
        pltpu.make_async_copy(bufs[b], out_hbm.at[pl.ds(base, CHUNK)],
                              ssems[b]).wait()

    for b in range(NBUF):
        start_gather(b, b)

    def body(g, _):
        for b in range(NBUF):
            wait_gather(b)
            start_store(g * NBUF + b, b)
        for b in range(NBUF):
            wait_store(b)
            start_gather((g + 1) * NBUF + b, b)
        return 0

    lax.fori_loop(0, NGRP - 1, body, 0)

    for b in range(NBUF):
        wait_gather(b)
        start_store((NGRP - 1) * NBUF + b, b)
    for b in range(NBUF):
        wait_store(b)


def kernel(x, embedding):
    x_flat = x.reshape(N).astype(jnp.int32)
    out = _gather_kernel(embedding, x_flat)
    return out.reshape(BATCH, SEQ, EMBED_DIM)


# speedup vs baseline: 7.4365x; 1.0256x over previous
import functools

import jax
import jax.numpy as jnp
from jax import lax
from jax.experimental import pallas as pl
from jax.experimental.pallas import tpu as pltpu
from jax.experimental.pallas import tpu_sc as plsc

BATCH = 1024
SEQ = 200
NUM_TOKENS = 100000
EMBED_DIM = 128

N = BATCH * SEQ
NW = 32
PER_W = N // NW
CHUNK = 400
NCHUNK = PER_W // CHUNK
NBUF = 2
NGRP = NCHUNK // NBUF

_mesh = plsc.VectorSubcoreMesh(core_axis_name="c", subcore_axis_name="s")


@functools.partial(
    pl.kernel,
    out_type=jax.ShapeDtypeStruct((N, EMBED_DIM), jnp.float32),
    mesh=_mesh,
    scratch_types=[
        pltpu.VMEM((PER_W,), jnp.int32),
        [pltpu.VMEM((CHUNK, EMBED_DIM), jnp.float32) for _ in range(NBUF)],
        [pltpu.SemaphoreType.DMA for _ in range(NBUF)],
        [pltpu.SemaphoreType.DMA for _ in range(NBUF)],
    ],
)
def _gather_kernel(table_hbm, idx_hbm, out_hbm, idx_v, bufs, gsems, ssems):
    wid = lax.axis_index("s") * 2 + lax.axis_index("c")
    base = wid * PER_W
    pltpu.sync_copy(idx_hbm.at[pl.ds(base, PER_W)], idx_v)

    def start_gather(j, b):
        idx_slice = idx_v.at[pl.ds(j * CHUNK, CHUNK)]
        pltpu.async_copy(table_hbm.at[idx_slice], bufs[b], gsems[b])

    def wait_gather(b):
        pltpu.make_async_copy(table_hbm.at[idx_v.at[pl.ds(0, CHUNK)]],
                              bufs[b], gsems[b]).wait()

    def start_store(j, b):
        pltpu.async_copy(bufs[b], out_hbm.at[pl.ds(base + j * CHUNK, CHUNK)],
                         ssems[b])

    def wait_store(b):
        pltpu.make_async_copy(bufs[b], out_hbm.at[pl.ds(base, CHUNK)],
                              ssems[b]).wait()

    for b in range(NBUF):
        start_gather(b, b)

    def body(g, _):
        for b in range(NBUF):
            wait_gather(b)
            start_store(g * NBUF + b, b)
        for b in range(NBUF):
            wait_store(b)
            start_gather((g + 1) * NBUF + b, b)
        return 0

    lax.fori_loop(0, NGRP - 1, body, 0)

    for b in range(NBUF):
        wait_gather(b)
        start_store((NGRP - 1) * NBUF + b, b)
    for b in range(NBUF):
        wait_store(b)


def kernel(x, embedding):
    x_flat = x.reshape(N).astype(jnp.int32)
    out = _gather_kernel(embedding, x_flat)
    return out.reshape(BATCH, SEQ, EMBED_DIM)


# speedup vs baseline: 7.6143x; 1.0239x over previous
import functools

import jax
import jax.numpy as jnp
from jax import lax
from jax.experimental import pallas as pl
from jax.experimental.pallas import tpu as pltpu
from jax.experimental.pallas import tpu_sc as plsc

BATCH = 1024
SEQ = 200
NUM_TOKENS = 100000
EMBED_DIM = 128

N = BATCH * SEQ
NW = 32
PER_W = N // NW
CHUNK = 200
NCHUNK = PER_W // CHUNK
NBUF = 4
NGRP = NCHUNK // NBUF

_mesh = plsc.VectorSubcoreMesh(core_axis_name="c", subcore_axis_name="s")


@functools.partial(
    pl.kernel,
    out_type=jax.ShapeDtypeStruct((N, EMBED_DIM), jnp.float32),
    mesh=_mesh,
    scratch_types=[
        pltpu.VMEM((PER_W,), jnp.int32),
        [pltpu.VMEM((CHUNK, EMBED_DIM), jnp.float32) for _ in range(NBUF)],
        [pltpu.SemaphoreType.DMA for _ in range(NBUF)],
        [pltpu.SemaphoreType.DMA for _ in range(NBUF)],
    ],
)
def _gather_kernel(table_hbm, idx_hbm, out_hbm, idx_v, bufs, gsems, ssems):
    wid = lax.axis_index("s") * 2 + lax.axis_index("c")
    base = wid * PER_W
    pltpu.sync_copy(idx_hbm.at[pl.ds(base, PER_W)], idx_v)

    def start_gather(j, b):
        idx_slice = idx_v.at[pl.ds(j * CHUNK, CHUNK)]
        pltpu.async_copy(table_hbm.at[idx_slice], bufs[b], gsems[b])

    def wait_gather(b):
        pltpu.make_async_copy(table_hbm.at[idx_v.at[pl.ds(0, CHUNK)]],
                              bufs[b], gsems[b]).wait()

    def start_store(j, b):
        pltpu.async_copy(bufs[b], out_hbm.at[pl.ds(base + j * CHUNK, CHUNK)],
                         ssems[b])

    def wait_store(b):
        pltpu.make_async_copy(bufs[b], out_hbm.at[pl.ds(base, CHUNK)],
                              ssems[b]).wait()

    for b in range(NBUF):
        start_gather(b, b)

    def body(g, _):
        for b in range(NBUF):
            wait_gather(b)
            start_store(g * NBUF + b, b)
        for b in range(NBUF):
            wait_store(b)
            start_gather((g + 1) * NBUF + b, b)
        return 0

    lax.fori_loop(0, NGRP - 1, body, 0)

    for b in range(NBUF):
        wait_gather(b)
        start_store((NGRP - 1) * NBUF + b, b)
    for b in range(NBUF):
        wait_store(b)


def kernel(x, embedding):
    x_flat = x.reshape(N).astype(jnp.int32)
    out = _gather_kernel(embedding, x_flat)
    return out.reshape(BATCH, SEQ, EMBED_DIM)


# speedup vs baseline: 7.8905x; 1.0363x over previous
import functools

import jax
import jax.numpy as jnp
from jax import lax
from jax.experimental import pallas as pl
from jax.experimental.pallas import tpu as pltpu
from jax.experimental.pallas import tpu_sc as plsc

BATCH = 1024
SEQ = 200
NUM_TOKENS = 100000
EMBED_DIM = 128

N = BATCH * SEQ
NW = 32
PER_W = N // NW
CHUNK = 80
NCHUNK = PER_W // CHUNK
NBUF = 8
NGRP = NCHUNK // NBUF

_mesh = plsc.VectorSubcoreMesh(core_axis_name="c", subcore_axis_name="s")


@functools.partial(
    pl.kernel,
    out_type=jax.ShapeDtypeStruct((N, EMBED_DIM), jnp.float32),
    mesh=_mesh,
    scratch_types=[
        pltpu.VMEM((PER_W,), jnp.int32),
        [pltpu.VMEM((CHUNK, EMBED_DIM), jnp.float32) for _ in range(NBUF)],
        [pltpu.SemaphoreType.DMA for _ in range(NBUF)],
        [pltpu.SemaphoreType.DMA for _ in range(NBUF)],
    ],
)
def _gather_kernel(table_hbm, idx_hbm, out_hbm, idx_v, bufs, gsems, ssems):
    wid = lax.axis_index("s") * 2 + lax.axis_index("c")
    base = wid * PER_W
    pltpu.sync_copy(idx_hbm.at[pl.ds(base, PER_W)], idx_v)

    def start_gather(j, b):
        idx_slice = idx_v.at[pl.ds(j * CHUNK, CHUNK)]
        pltpu.async_copy(table_hbm.at[idx_slice], bufs[b], gsems[b])

    def wait_gather(b):
        pltpu.make_async_copy(table_hbm.at[idx_v.at[pl.ds(0, CHUNK)]],
                              bufs[b], gsems[b]).wait()

    def start_store(j, b):
        pltpu.async_copy(bufs[b], out_hbm.at[pl.ds(base + j * CHUNK, CHUNK)],
                         ssems[b])

    def wait_store(b):
        pltpu.make_async_copy(bufs[b], out_hbm.at[pl.ds(base, CHUNK)],
                              ssems[b]).wait()

    for b in range(NBUF):
        start_gather(b, b)

    def body(g, _):
        for b in range(NBUF):
            wait_gather(b)
            start_store(g * NBUF + b, b)
        for b in range(NBUF):
            wait_store(b)
            start_gather((g + 1) * NBUF + b, b)
        return 0

    lax.fori_loop(0, NGRP - 1, body, 0)

    for b in range(NBUF):
        wait_gather(b)
        start_store((NGRP - 1) * NBUF + b, b)
    for b in range(NBUF):
        wait_store(b)


def kernel(x, embedding):
    x_flat = x.reshape(N).astype(jnp.int32)
    out = _gather_kernel(embedding, x_flat)
    return out.reshape(BATCH, SEQ, EMBED_DIM)


# speedup vs baseline: 7.9355x; 1.0057x over previous
import functools

import jax
import jax.numpy as jnp
from jax import lax
from jax.experimental import pallas as pl
from jax.experimental.pallas import tpu as pltpu
from jax.experimental.pallas import tpu_sc as plsc

BATCH = 1024
SEQ = 200
NUM_TOKENS = 100000
EMBED_DIM = 128

N = BATCH * SEQ
NW = 32
PER_W = N // NW
CHUNK = 64
NCHUNK = PER_W // CHUNK
NBUF = 10
NGRP = NCHUNK // NBUF

_mesh = plsc.VectorSubcoreMesh(core_axis_name="c", subcore_axis_name="s")


@functools.partial(
    pl.kernel,
    out_type=jax.ShapeDtypeStruct((N, EMBED_DIM), jnp.float32),
    mesh=_mesh,
    scratch_types=[
        pltpu.VMEM((PER_W,), jnp.int32),
        [pltpu.VMEM((CHUNK, EMBED_DIM), jnp.float32) for _ in range(NBUF)],
        [pltpu.SemaphoreType.DMA for _ in range(NBUF)],
        [pltpu.SemaphoreType.DMA for _ in range(NBUF)],
    ],
)
def _gather_kernel(table_hbm, idx_hbm, out_hbm, idx_v, bufs, gsems, ssems):
    wid = lax.axis_index("s") * 2 + lax.axis_index("c")
    base = wid * PER_W
    pltpu.sync_copy(idx_hbm.at[pl.ds(base, PER_W)], idx_v)

    def start_gather(j, b):
        idx_slice = idx_v.at[pl.ds(j * CHUNK, CHUNK)]
        pltpu.async_copy(table_hbm.at[idx_slice], bufs[b], gsems[b])

    def wait_gather(b):
        pltpu.make_async_copy(table_hbm.at[idx_v.at[pl.ds(0, CHUNK)]],
                              bufs[b], gsems[b]).wait()

    def start_store(j, b):
        pltpu.async_copy(bufs[b], out_hbm.at[pl.ds(base + j * CHUNK, CHUNK)],
                         ssems[b])

    def wait_store(b):
        pltpu.make_async_copy(bufs[b], out_hbm.at[pl.ds(base, CHUNK)],
                              ssems[b]).wait()

    for b in range(NBUF):
        start_gather(b, b)

    def body(g, _):
        for b in range(NBUF):
            wait_gather(b)
            start_store(g * NBUF + b, b)
        for b in range(NBUF):
            wait_store(b)
            start_gather((g + 1) * NBUF + b, b)
        return 0

    lax.fori_loop(0, NGRP - 1, body, 0)

    for b in range(NBUF):
        wait_gather(b)
        start_store((NGRP - 1) * NBUF + b, b)
    for b in range(NBUF):
        wait_store(b)


def kernel(x, embedding):
    x_flat = x.reshape(N).astype(jnp.int32)
    out = _gather_kernel(embedding, x_flat)
    return out.reshape(BATCH, SEQ, EMBED_DIM)
